# baseline (speedup 1.0000x reference)
import functools

import jax
import jax.numpy as jnp
from jax import lax
from jax.experimental import pallas as pl
from jax.experimental.pallas import tpu as pltpu
from jax.experimental.pallas import tpu_sc as plsc

EPS = 1e-12
NC = 2
NS = 16
NW = NC * NS


def _sc_gather(word_emb, ids, n_tokens, hidden, dtype):
    per_w = n_tokens // NW
    CH = 32
    NBUF = 4
    AHEAD = 2
    nch = per_w // CH
    ids3 = ids.reshape(NW, nch, CH)

    mesh = plsc.VectorSubcoreMesh(core_axis_name="c", subcore_axis_name="s")

    @functools.partial(
        pl.kernel,
        mesh=mesh,
        out_type=jax.ShapeDtypeStruct((n_tokens, hidden), dtype),
        scratch_types=[
            pltpu.VMEM((nch, CH), jnp.int32),
            [pltpu.VMEM((CH, hidden), dtype)] * NBUF,
            [pltpu.SemaphoreType.DMA] * NBUF,
            [pltpu.SemaphoreType.DMA] * NBUF,
        ],
    )
    def gather_k(table_hbm, ids_hbm, out_hbm, idx_v, rows, gsems, psems):
        wid = lax.axis_index("s") * NC + lax.axis_index("c")
        base = wid * per_w

        def gstart(c, b):
            pltpu.make_async_copy(table_hbm.at[idx_v.at[c]], rows[b], gsems[b]).start()

        def gwait(b):
            pltpu.make_async_copy(table_hbm.at[idx_v.at[0]], rows[b], gsems[b]).wait()

        def pstart(c, b):
            row_start = pl.multiple_of(base + c * CH, CH)
            pltpu.make_async_copy(rows[b], out_hbm.at[pl.ds(row_start, CH)], psems[b]).start()

        def pwait(b):
            pltpu.make_async_copy(rows[b], out_hbm.at[pl.ds(base, CH)], psems[b]).wait()

        pltpu.sync_copy(ids_hbm.at[wid], idx_v)
        for c0 in range(AHEAD):
            gstart(c0, c0 % NBUF)

        ngrp = nch // NBUF

        def group(i, first=False, last=False):
            for b in range(NBUF):
                c = i * NBUF + b
                gwait(b)
                pstart(c, b)
                if (not last) or (b < AHEAD):
                    bn = (b + AHEAD) % NBUF
                    if not (first and b < AHEAD):
                        pwait(bn)
                    gstart(c + AHEAD, bn)

        group(0, first=True)

        def body(i, _):
            group(i)
            return 0

        lax.fori_loop(1, ngrp - 1, body, 0)
        group(ngrp - 1, last=True)
        for b in range(NBUF):
            pwait(b)

    return gather_k(word_emb, ids3)


def _tc_fused_slice(we3, token_type_ids, pe, tok_emb, gamma2, beta2,
                    prev_out, slice_idx, b_total):
    Bs, S, Hd = we3.shape
    BB = 8
    grid = (Bs // BB,)
    off = slice_idx * (Bs // BB)

    def body(*refs):
        if slice_idx == 0:
            we_ref, tt_ref, pe_ref, tok_ref, g_ref, b_ref, out_ref = refs
        else:
            we_ref, tt_ref, pe_ref, tok_ref, g_ref, b_ref, _prev, out_ref = refs
        we = we_ref[...].astype(jnp.float32)
        tt = tt_ref[...].astype(jnp.float32)[..., None]
        pos = pe_ref[...][None]
        tok0 = tok_ref[0][None, None, :]
        tokd = (tok_ref[1] - tok_ref[0])[None, None, :]
        emb = we + pos + tok0 + tt * tokd
        mu = jnp.mean(emb, axis=-1, keepdims=True)
        cen = emb - mu
        var = jnp.mean(cen * cen, axis=-1, keepdims=True)
        out_ref[...] = cen * lax.rsqrt(var + EPS) * g_ref[0][None, None, :] + b_ref[0][None, None, :]

    in_specs = [
        pl.BlockSpec((BB, S, Hd), lambda i: (i, 0, 0)),
        pl.BlockSpec((BB, S), lambda i: (i, 0)),
        pl.BlockSpec((S, Hd), lambda i: (0, 0)),
        pl.BlockSpec((2, Hd), lambda i: (0, 0)),
        pl.BlockSpec((1, Hd), lambda i: (0, 0)),
        pl.BlockSpec((1, Hd), lambda i: (0, 0)),
    ]
    args = [we3, token_type_ids, pe, tok_emb, gamma2, beta2]
    aliases = {}
    if slice_idx > 0:
        in_specs.append(pl.BlockSpec(memory_space=pl.ANY))
        args.append(prev_out)
        aliases = {6: 0}

    return pl.pallas_call(
        body,
        grid=grid,
        in_specs=in_specs,
        out_specs=pl.BlockSpec((BB, S, Hd), lambda i: (i + off, 0, 0)),
        out_shape=jax.ShapeDtypeStruct((b_total, S, Hd), jnp.float32),
        input_output_aliases=aliases,
    )(*args)


def _tc_mask(attention_mask):
    B, S = attention_mask.shape
    BB = 128
    grid = (B // BB,)

    def body(am_ref, mask_ref):
        am = am_ref[...].astype(jnp.float32)
        mask_ref[...] = ((1.0 - am) * -10000.0)[:, None, :]

    return pl.pallas_call(
        body,
        grid=grid,
        in_specs=[pl.BlockSpec((BB, S), lambda i: (i, 0))],
        out_specs=pl.BlockSpec((BB, 1, S), lambda i: (i, 0, 0)),
        out_shape=jax.ShapeDtypeStruct((B, 1, S), jnp.float32),
    )(attention_mask)


def kernel(input_ids, attention_mask, token_type_ids, word_emb, pos_emb, tok_emb, gamma, beta):
    B, S = input_ids.shape
    V, Hd = word_emb.shape
    n = B * S
    ids = input_ids.reshape(-1).astype(jnp.int32)
    NSLICE = 4
    bs = B // NSLICE
    ns = n // NSLICE
    tt = token_type_ids.astype(jnp.int32)
    pe = pos_emb[:S]
    gamma2 = gamma.reshape(1, Hd)
    beta2 = beta.reshape(1, Hd)
    word_emb_h = lax.bitcast_convert_type(
        word_emb.astype(jnp.bfloat16).reshape(V, Hd // 2, 2), jnp.int32)
    we_slices = [_sc_gather(word_emb_h, ids[i * ns:(i + 1) * ns], ns, Hd // 2, jnp.int32)
                 for i in range(NSLICE)]
    we_slices = [
        lax.bitcast_convert_type(w, jnp.bfloat16).reshape(bs, S, Hd)
        for w in we_slices
    ]
    mask = _tc_mask(attention_mask.astype(jnp.int32))
    out = None
    for i, we in enumerate(we_slices):
        out = _tc_fused_slice(we,
                              tt[i * bs:(i + 1) * bs],
                              pe, tok_emb, gamma2, beta2,
                              out, i, B)
    return (out, mask)


# speedup vs baseline: 4.4308x; 4.4308x over previous
import functools

import jax
import jax.numpy as jnp
from jax import lax
from jax.experimental import pallas as pl
from jax.experimental.pallas import tpu as pltpu
from jax.experimental.pallas import tpu_sc as plsc

EPS = 1e-12
NC = 2
NS = 16
NW = NC * NS


def _sc_gather(word_emb, ids, n_tokens, hidden, dtype):
    per_w = n_tokens // NW
    CH = 32
    NBUF = 4
    AHEAD = 2
    nch = per_w // CH
    ids3 = ids.reshape(NW, nch, CH)

    mesh = plsc.VectorSubcoreMesh(core_axis_name="c", subcore_axis_name="s")

    @functools.partial(
        pl.kernel,
        mesh=mesh,
        out_type=jax.ShapeDtypeStruct((n_tokens, hidden), dtype),
        scratch_types=[
            pltpu.VMEM((nch, CH), jnp.int32),
            [pltpu.VMEM((CH, hidden), dtype)] * NBUF,
            [pltpu.SemaphoreType.DMA] * NBUF,
            [pltpu.SemaphoreType.DMA] * NBUF,
        ],
    )
    def gather_k(table_hbm, ids_hbm, out_hbm, idx_v, rows, gsems, psems):
        wid = lax.axis_index("s") * NC + lax.axis_index("c")
        base = wid * per_w

        def gstart(c, b):
            pltpu.make_async_copy(table_hbm.at[idx_v.at[c]], rows[b], gsems[b]).start()

        def gwait(b):
            pltpu.make_async_copy(table_hbm.at[idx_v.at[0]], rows[b], gsems[b]).wait()

        def pstart(c, b):
            row_start = pl.multiple_of(base + c * CH, CH)
            pltpu.make_async_copy(rows[b], out_hbm.at[pl.ds(row_start, CH)], psems[b]).start()

        def pwait(b):
            pltpu.make_async_copy(rows[b], out_hbm.at[pl.ds(base, CH)], psems[b]).wait()

        pltpu.sync_copy(ids_hbm.at[wid], idx_v)
        for c0 in range(AHEAD):
            gstart(c0, c0 % NBUF)

        ngrp = nch // NBUF

        def group(i, first=False, last=False):
            for b in range(NBUF):
                c = i * NBUF + b
                gwait(b)
                pstart(c, b)
                if (not last) or (b < AHEAD):
                    bn = (b + AHEAD) % NBUF
                    if not (first and b < AHEAD):
                        pwait(bn)
                    gstart(c + AHEAD, bn)

        group(0, first=True)

        def body(i, _):
            group(i)
            return 0

        lax.fori_loop(1, ngrp - 1, body, 0)
        group(ngrp - 1, last=True)
        for b in range(NBUF):
            pwait(b)

    return gather_k(word_emb, ids3)


def _tc_fused_slice(we3, token_type_ids, pe, tok_emb, gamma2, beta2,
                    prev_out, slice_idx, b_total):
    Bs, S, Hh = we3.shape
    Hd = Hh * 2
    BB = 8
    grid = (Bs // BB,)
    off = slice_idx * (Bs // BB)

    def body(*refs):
        if slice_idx == 0:
            we_ref, tt_ref, pe_ref, tok_ref, g_ref, b_ref, out_ref = refs
        else:
            we_ref, tt_ref, pe_ref, tok_ref, g_ref, b_ref, _prev, out_ref = refs
        x = we_ref[...]
        lo = lax.bitcast_convert_type(x << 16, jnp.float32)
        hi = lax.bitcast_convert_type(x & jnp.int32(-65536), jnp.float32)
        tt = tt_ref[...].astype(jnp.float32)[..., None]
        tok0 = tok_ref[0][None, None, :]
        tokd = (tok_ref[1] - tok_ref[0])[None, None, :]
        emb_lo = lo + pe_ref[:, :Hh][None] + tok0[..., :Hh] + tt * tokd[..., :Hh]
        emb_hi = hi + pe_ref[:, Hh:][None] + tok0[..., Hh:] + tt * tokd[..., Hh:]
        s = (jnp.sum(emb_lo, axis=-1, keepdims=True)
             + jnp.sum(emb_hi, axis=-1, keepdims=True))
        mu = s * (1.0 / Hd)
        cen_lo = emb_lo - mu
        cen_hi = emb_hi - mu
        sq = (jnp.sum(cen_lo * cen_lo, axis=-1, keepdims=True)
              + jnp.sum(cen_hi * cen_hi, axis=-1, keepdims=True))
        rstd = lax.rsqrt(sq * (1.0 / Hd) + EPS)
        g = g_ref[0][None, None, :]
        b = b_ref[0][None, None, :]
        out_ref[:, :, :Hh] = cen_lo * rstd * g[..., :Hh] + b[..., :Hh]
        out_ref[:, :, Hh:] = cen_hi * rstd * g[..., Hh:] + b[..., Hh:]

    in_specs = [
        pl.BlockSpec((BB, S, Hh), lambda i: (i, 0, 0)),
        pl.BlockSpec((BB, S), lambda i: (i, 0)),
        pl.BlockSpec((S, Hd), lambda i: (0, 0)),
        pl.BlockSpec((2, Hd), lambda i: (0, 0)),
        pl.BlockSpec((1, Hd), lambda i: (0, 0)),
        pl.BlockSpec((1, Hd), lambda i: (0, 0)),
    ]
    args = [we3, token_type_ids, pe, tok_emb, gamma2, beta2]
    aliases = {}
    if slice_idx > 0:
        in_specs.append(pl.BlockSpec(memory_space=pl.ANY))
        args.append(prev_out)
        aliases = {6: 0}

    return pl.pallas_call(
        body,
        grid=grid,
        in_specs=in_specs,
        out_specs=pl.BlockSpec((BB, S, Hd), lambda i: (i + off, 0, 0)),
        out_shape=jax.ShapeDtypeStruct((b_total, S, Hd), jnp.float32),
        input_output_aliases=aliases,
    )(*args)


def _tc_mask(attention_mask):
    B, S = attention_mask.shape
    BB = 128
    grid = (B // BB,)

    def body(am_ref, mask_ref):
        am = am_ref[...].astype(jnp.float32)
        mask_ref[...] = ((1.0 - am) * -10000.0)[:, None, :]

    return pl.pallas_call(
        body,
        grid=grid,
        in_specs=[pl.BlockSpec((BB, S), lambda i: (i, 0))],
        out_specs=pl.BlockSpec((BB, 1, S), lambda i: (i, 0, 0)),
        out_shape=jax.ShapeDtypeStruct((B, 1, S), jnp.float32),
    )(attention_mask)


def kernel(input_ids, attention_mask, token_type_ids, word_emb, pos_emb, tok_emb, gamma, beta):
    B, S = input_ids.shape
    V, Hd = word_emb.shape
    n = B * S
    ids = input_ids.reshape(-1).astype(jnp.int32)
    NSLICE = 4
    bs = B // NSLICE
    ns = n // NSLICE
    tt = token_type_ids.astype(jnp.int32)
    pe = pos_emb[:S]
    gamma2 = gamma.reshape(1, Hd)
    beta2 = beta.reshape(1, Hd)
    Hh = Hd // 2
    wb = word_emb.astype(jnp.bfloat16)
    word_emb_h = lax.bitcast_convert_type(
        jnp.stack([wb[:, :Hh], wb[:, Hh:]], axis=-1), jnp.int32)
    we_slices = [
        _sc_gather(word_emb_h, ids[i * ns:(i + 1) * ns], ns, Hh, jnp.int32)
        .reshape(bs, S, Hh)
        for i in range(NSLICE)
    ]
    mask = _tc_mask(attention_mask.astype(jnp.int32))
    out = None
    for i, we in enumerate(we_slices):
        out = _tc_fused_slice(we,
                              tt[i * bs:(i + 1) * bs],
                              pe, tok_emb, gamma2, beta2,
                              out, i, B)
    return (out, mask)
